# Initial kernel scaffold; baseline (speedup 1.0000x reference)
import jax
import jax.numpy as jnp
from jax.experimental import pallas as pl


def kernel(x, edge_index, edge_attr, params):
    raise NotImplementedError("write your pallas kernel here")


# speedup vs baseline: 2.8731x; 2.8731x over previous
import functools

import jax
import jax.numpy as jnp
from jax import lax
from jax.experimental import pallas as pl
from jax.experimental.pallas import tpu as pltpu
from jax.experimental.pallas import tpu_sc as plsc

N = 10000
E = 320000
H = 64

NC = 2
NS = 16
NW = NC * NS
EW = E // NW
CH = 1000
NCHUNK = EW // CH
ROWS_PER_TILE = N // NS


@functools.lru_cache(maxsize=None)
def _make_gather():
    mesh = plsc.VectorSubcoreMesh(core_axis_name="c", subcore_axis_name="s")

    @functools.partial(
        pl.kernel,
        mesh=mesh,
        out_type=(jax.ShapeDtypeStruct((E, H), jnp.float32),
                  jax.ShapeDtypeStruct((E, H), jnp.float32)),
        scratch_types=[pltpu.VMEM((CH,), jnp.int32),
                       pltpu.VMEM((CH, H), jnp.float32),
                       pltpu.SemaphoreType.DMA],
        compiler_params=pltpu.CompilerParams(use_tc_tiling_on_sc=False),
    )
    def gather_k(a_hbm, b_hbm, src_hbm, dst_hbm, p1_hbm, p2_hbm,
                 idx_v, rows_v, sem):
        wid = lax.axis_index("s") * NC + lax.axis_index("c")
        base0 = wid * EW

        @pl.loop(0, NCHUNK)
        def _(i):
            base = base0 + i * CH
            pltpu.sync_copy(src_hbm.at[pl.ds(base, CH)], idx_v)
            pltpu.async_copy(a_hbm.at[idx_v], rows_v, sem).wait()
            pltpu.sync_copy(rows_v, p1_hbm.at[pl.ds(base, CH)])
            pltpu.sync_copy(dst_hbm.at[pl.ds(base, CH)], idx_v)
            pltpu.async_copy(b_hbm.at[idx_v], rows_v, sem).wait()
            pltpu.sync_copy(rows_v, p2_hbm.at[pl.ds(base, CH)])

    return gather_k


@functools.lru_cache(maxsize=None)
def _make_scatter(d):
    mesh = plsc.VectorSubcoreMesh(core_axis_name="c", subcore_axis_name="s")
    epc = E // NC
    ept = epc // NS
    nchunk = ept // CH

    @functools.partial(
        pl.kernel,
        mesh=mesh,
        out_type=jax.ShapeDtypeStruct((NC, N, d), jnp.float32),
        scratch_types=[pltpu.VMEM((CH,), jnp.int32),
                       pltpu.VMEM((CH, d), jnp.float32),
                       pltpu.VMEM_SHARED((N, d), jnp.float32),
                       pltpu.SemaphoreType.DMA],
        compiler_params=pltpu.CompilerParams(use_tc_tiling_on_sc=False),
    )
    def scatter_k(e_hbm, dst_hbm, zeros_hbm, out_hbm,
                  idx_v, rows_v, acc_sh, sem):
        c = lax.axis_index("c")
        s = lax.axis_index("s")
        rbase = s * ROWS_PER_TILE
        pltpu.sync_copy(zeros_hbm.at[pl.ds(rbase, ROWS_PER_TILE)],
                        acc_sh.at[pl.ds(rbase, ROWS_PER_TILE)])
        plsc.subcore_barrier()

        ebase0 = c * epc + s * ept

        @pl.loop(0, nchunk)
        def _(i):
            base = ebase0 + i * CH
            pltpu.sync_copy(dst_hbm.at[pl.ds(base, CH)], idx_v)
            pltpu.sync_copy(e_hbm.at[pl.ds(base, CH)], rows_v)
            pltpu.sync_copy(rows_v, acc_sh.at[idx_v], add=True)

        plsc.subcore_barrier()
        pltpu.sync_copy(acc_sh.at[pl.ds(rbase, ROWS_PER_TILE)],
                        out_hbm.at[c, pl.ds(rbase, ROWS_PER_TILE)])

    return scatter_k


BN = 2000
BE = 8000


def _relu(v):
    return jnp.maximum(v, 0.0)


def _node_pre_body(h_ref, wns_ref, bns_ref, wne1_ref, wne2_ref,
                   hn_ref, a_ref, b_ref):
    h = h_ref[...]
    hn_ref[...] = _relu(
        jnp.dot(h, wns_ref[...], preferred_element_type=jnp.float32)
        + bns_ref[...])
    a_ref[...] = jnp.dot(h, wne1_ref[...], preferred_element_type=jnp.float32)
    b_ref[...] = jnp.dot(h, wne2_ref[...], preferred_element_type=jnp.float32)


def _node_pre_hn_body(h_ref, wns_ref, bns_ref, hn_ref):
    hn_ref[...] = _relu(
        jnp.dot(h_ref[...], wns_ref[...], preferred_element_type=jnp.float32)
        + bns_ref[...])


def _node_post_body(agg0_ref, agg1_ref, hn_ref, wen_ref, ben_ref,
                    wno1_ref, wno2_ref, bno_ref, h_ref):
    agg = agg0_ref[...] + agg1_ref[...]
    en = _relu(jnp.dot(agg, wen_ref[...], preferred_element_type=jnp.float32)
               + ben_ref[...])
    h_ref[...] = _relu(
        jnp.dot(hn_ref[...], wno1_ref[...], preferred_element_type=jnp.float32)
        + jnp.dot(en, wno2_ref[...], preferred_element_type=jnp.float32)
        + bno_ref[...])


def _edge_body(e_ref, p1_ref, p2_ref, wes_ref, bes_ref, bne_ref,
               weo1_ref, weo2_ref, beo_ref, out_ref):
    ee = _relu(jnp.dot(e_ref[...], wes_ref[...],
                       preferred_element_type=jnp.float32) + bes_ref[...])
    ne = _relu(p1_ref[...] + p2_ref[...] + bne_ref[...])
    out_ref[...] = _relu(
        jnp.dot(ee, weo1_ref[...], preferred_element_type=jnp.float32)
        + jnp.dot(ne, weo2_ref[...], preferred_element_type=jnp.float32)
        + beo_ref[...])


def _wspec(shape):
    nd = len(shape)
    return pl.BlockSpec(shape, lambda i: (0,) * nd)


def _node_pre(h, wns, bns, wne1, wne2):
    nin = h.shape[1]
    grid = (N // BN,)
    return pl.pallas_call(
        _node_pre_body,
        grid=grid,
        in_specs=[pl.BlockSpec((BN, nin), lambda i: (i, 0)),
                  _wspec((nin, H)), _wspec((1, H)),
                  _wspec((nin, H)), _wspec((nin, H))],
        out_specs=[pl.BlockSpec((BN, H), lambda i: (i, 0))] * 3,
        out_shape=[jax.ShapeDtypeStruct((N, H), jnp.float32)] * 3,
    )(h, wns, bns, wne1, wne2)


def _node_pre_hn(h, wns, bns):
    nin = h.shape[1]
    grid = (N // BN,)
    return pl.pallas_call(
        _node_pre_hn_body,
        grid=grid,
        in_specs=[pl.BlockSpec((BN, nin), lambda i: (i, 0)),
                  _wspec((nin, H)), _wspec((1, H))],
        out_specs=pl.BlockSpec((BN, H), lambda i: (i, 0)),
        out_shape=jax.ShapeDtypeStruct((N, H), jnp.float32),
    )(h, wns, bns)


def _node_post(agg0, agg1, hn, wen, ben, wno1, wno2, bno):
    ein = agg0.shape[1]
    grid = (N // BN,)
    return pl.pallas_call(
        _node_post_body,
        grid=grid,
        in_specs=[pl.BlockSpec((BN, ein), lambda i: (i, 0)),
                  pl.BlockSpec((BN, ein), lambda i: (i, 0)),
                  pl.BlockSpec((BN, H), lambda i: (i, 0)),
                  _wspec((ein, H)), _wspec((1, H)),
                  _wspec((H, H)), _wspec((H, H)), _wspec((1, H))],
        out_specs=pl.BlockSpec((BN, H), lambda i: (i, 0)),
        out_shape=jax.ShapeDtypeStruct((N, H), jnp.float32),
    )(agg0, agg1, hn, wen, ben, wno1, wno2, bno)


def _edge_update(e, p1, p2, wes, bes, bne, weo1, weo2, beo):
    ein = e.shape[1]
    grid = (E // BE,)
    return pl.pallas_call(
        _edge_body,
        grid=grid,
        in_specs=[pl.BlockSpec((BE, ein), lambda i: (i, 0)),
                  pl.BlockSpec((BE, H), lambda i: (i, 0)),
                  pl.BlockSpec((BE, H), lambda i: (i, 0)),
                  _wspec((ein, H)), _wspec((1, H)), _wspec((1, H)),
                  _wspec((H, H)), _wspec((H, H)), _wspec((1, H))],
        out_specs=pl.BlockSpec((BE, H), lambda i: (i, 0)),
        out_shape=jax.ShapeDtypeStruct((E, H), jnp.float32),
    )(e, p1, p2, wes, bes, bne, weo1, weo2, beo)


def kernel(x, edge_index, edge_attr, params):
    src = edge_index[0]
    dst = edge_index[1]
    h = x
    e = edge_attr
    n_layers = len(params)
    for l, p in enumerate(params):
        nin = h.shape[1]
        ein = e.shape[1]
        last = l == n_layers - 1
        r1 = lambda v: v.reshape(1, -1)

        zeros = jnp.zeros((N, ein), jnp.float32)
        aggp = _make_scatter(ein)(e, dst, zeros)

        if last:
            hn = _node_pre_hn(h, p['W_ns'], r1(p['b_ns']))
        else:
            hn, a, b = _node_pre(h, p['W_ns'], r1(p['b_ns']),
                                 p['W_ne'][:nin], p['W_ne'][nin:])
            p1, p2 = _make_gather()(a, b, src, dst)
            e = _edge_update(e, p1, p2, p['W_es'], r1(p['b_es']),
                             r1(p['b_ne']), p['W_eo'][:H], p['W_eo'][H:],
                             r1(p['b_eo']))

        h = _node_post(aggp[0], aggp[1], hn, p['W_en'], r1(p['b_en']),
                       p['W_no'][:H], p['W_no'][H:], r1(p['b_no']))
    return h


# speedup vs baseline: 3.6927x; 1.2853x over previous
import functools

import jax
import jax.numpy as jnp
from jax import lax
from jax.experimental import pallas as pl
from jax.experimental.pallas import tpu as pltpu
from jax.experimental.pallas import tpu_sc as plsc

N = 10000
E = 320000
H = 64

NC = 2
NS = 16
NW = NC * NS
EW = E // NW
ROWS_PER_TILE = N // NS


GCH = 400
GNCHUNK = EW // GCH


@functools.lru_cache(maxsize=None)
def _make_gather():
    mesh = plsc.VectorSubcoreMesh(core_axis_name="c", subcore_axis_name="s")

    @functools.partial(
        pl.kernel,
        mesh=mesh,
        out_type=jax.ShapeDtypeStruct((E, H), jnp.float32),
        scratch_types=[pltpu.VMEM((GCH,), jnp.int32),
                       pltpu.VMEM((GCH,), jnp.int32),
                       pltpu.VMEM((GCH,), jnp.int32),
                       pltpu.VMEM((GCH,), jnp.int32),
                       pltpu.VMEM((GCH, H), jnp.float32),
                       pltpu.VMEM((GCH, H), jnp.float32),
                       pltpu.VMEM((GCH, H), jnp.float32),
                       pltpu.VMEM((GCH, H), jnp.float32),
                       pltpu.SemaphoreType.DMA,
                       pltpu.SemaphoreType.DMA,
                       pltpu.SemaphoreType.DMA,
                       pltpu.SemaphoreType.DMA],
        compiler_params=pltpu.CompilerParams(use_tc_tiling_on_sc=False),
    )
    def gather_k(a_hbm, b_hbm, src_hbm, dst_hbm, p_hbm,
                 idxa0, idxb0, idxa1, idxb1, bufa0, bufb0, bufa1, bufb1,
                 sa0, sb0, sa1, sb1):
        wid = lax.axis_index("s") * NC + lax.axis_index("c")
        base0 = wid * EW

        def vadd_out(bufa, bufb, base):
            @pl.loop(0, GCH)
            def _(r):
                for jj in range(H // 16):
                    sl = pl.ds(jj * 16, 16)
                    plsc.addupdate(bufa.at[r, sl], bufb[r, sl])

            pltpu.sync_copy(bufa, p_hbm.at[pl.ds(base, GCH)])

        @pl.loop(0, GNCHUNK - (GNCHUNK % 2), step=2)
        def _(i):
            b0 = base0 + i * GCH
            b1 = b0 + GCH
            pltpu.sync_copy(src_hbm.at[pl.ds(b0, GCH)], idxa0)
            pltpu.sync_copy(dst_hbm.at[pl.ds(b0, GCH)], idxb0)
            pltpu.sync_copy(src_hbm.at[pl.ds(b1, GCH)], idxa1)
            pltpu.sync_copy(dst_hbm.at[pl.ds(b1, GCH)], idxb1)
            da0 = pltpu.async_copy(a_hbm.at[idxa0], bufa0, sa0)
            db0 = pltpu.async_copy(b_hbm.at[idxb0], bufb0, sb0)
            da1 = pltpu.async_copy(a_hbm.at[idxa1], bufa1, sa1)
            db1 = pltpu.async_copy(b_hbm.at[idxb1], bufb1, sb1)
            da0.wait()
            db0.wait()
            vadd_out(bufa0, bufb0, b0)
            da1.wait()
            db1.wait()
            vadd_out(bufa1, bufb1, b1)

        if GNCHUNK % 2:
            bt = base0 + (GNCHUNK - 1) * GCH
            pltpu.sync_copy(src_hbm.at[pl.ds(bt, GCH)], idxa0)
            pltpu.sync_copy(dst_hbm.at[pl.ds(bt, GCH)], idxb0)
            da0 = pltpu.async_copy(a_hbm.at[idxa0], bufa0, sa0)
            db0 = pltpu.async_copy(b_hbm.at[idxb0], bufb0, sb0)
            da0.wait()
            db0.wait()
            vadd_out(bufa0, bufb0, bt)

    return gather_k


SCH = 400


@functools.lru_cache(maxsize=None)
def _make_scatter(d):
    mesh = plsc.VectorSubcoreMesh(core_axis_name="c", subcore_axis_name="s")
    epc = E // NC
    ept = epc // NS
    nchunk = ept // SCH

    @functools.partial(
        pl.kernel,
        mesh=mesh,
        out_type=jax.ShapeDtypeStruct((NC, N, d), jnp.float32),
        scratch_types=[pltpu.VMEM((SCH,), jnp.int32),
                       pltpu.VMEM((SCH,), jnp.int32),
                       pltpu.VMEM((SCH, d), jnp.float32),
                       pltpu.VMEM((SCH, d), jnp.float32),
                       pltpu.VMEM_SHARED((N, d), jnp.float32),
                       pltpu.SemaphoreType.DMA,
                       pltpu.SemaphoreType.DMA,
                       pltpu.SemaphoreType.DMA,
                       pltpu.SemaphoreType.DMA],
        compiler_params=pltpu.CompilerParams(use_tc_tiling_on_sc=False),
    )
    def scatter_k(e_hbm, dst_hbm, zeros_hbm, out_hbm,
                  idx0, idx1, rows0, rows1, acc_sh, si0, si1, sr0, sr1):
        c = lax.axis_index("c")
        s = lax.axis_index("s")
        rbase = s * ROWS_PER_TILE
        pltpu.sync_copy(zeros_hbm.at[pl.ds(rbase, ROWS_PER_TILE)],
                        acc_sh.at[pl.ds(rbase, ROWS_PER_TILE)])
        plsc.subcore_barrier()

        ebase0 = c * epc + s * ept

        @pl.loop(0, nchunk - (nchunk % 2), step=2)
        def _(i):
            b0 = ebase0 + i * SCH
            b1 = b0 + SCH
            di0 = pltpu.async_copy(dst_hbm.at[pl.ds(b0, SCH)], idx0, si0)
            dr0 = pltpu.async_copy(e_hbm.at[pl.ds(b0, SCH)], rows0, sr0)
            di1 = pltpu.async_copy(dst_hbm.at[pl.ds(b1, SCH)], idx1, si1)
            dr1 = pltpu.async_copy(e_hbm.at[pl.ds(b1, SCH)], rows1, sr1)
            di0.wait()
            dr0.wait()
            pltpu.sync_copy(rows0, acc_sh.at[idx0], add=True)
            di1.wait()
            dr1.wait()
            pltpu.sync_copy(rows1, acc_sh.at[idx1], add=True)

        if nchunk % 2:
            bt = ebase0 + (nchunk - 1) * SCH
            di0 = pltpu.async_copy(dst_hbm.at[pl.ds(bt, SCH)], idx0, si0)
            dr0 = pltpu.async_copy(e_hbm.at[pl.ds(bt, SCH)], rows0, sr0)
            di0.wait()
            dr0.wait()
            pltpu.sync_copy(rows0, acc_sh.at[idx0], add=True)

        plsc.subcore_barrier()
        pltpu.sync_copy(acc_sh.at[pl.ds(rbase, ROWS_PER_TILE)],
                        out_hbm.at[c, pl.ds(rbase, ROWS_PER_TILE)])

    return scatter_k


BN = 2000
BE = 8000


def _relu(v):
    return jnp.maximum(v, 0.0)


def _node_pre_body(h_ref, wns_ref, bns_ref, wne1_ref, wne2_ref,
                   hn_ref, a_ref, b_ref):
    h = h_ref[...]
    hn_ref[...] = _relu(
        jnp.dot(h, wns_ref[...], preferred_element_type=jnp.float32)
        + bns_ref[...])
    a_ref[...] = jnp.dot(h, wne1_ref[...], preferred_element_type=jnp.float32)
    b_ref[...] = jnp.dot(h, wne2_ref[...], preferred_element_type=jnp.float32)


def _node_merged_body(agg0_ref, agg1_ref, hn_ref, wen_ref, ben_ref,
                      wno1_ref, wno2_ref, bno_ref,
                      wns_ref, bns_ref, wne1_ref, wne2_ref,
                      hn2_ref, a_ref, b_ref):
    agg = agg0_ref[...] + agg1_ref[...]
    en = _relu(jnp.dot(agg, wen_ref[...], preferred_element_type=jnp.float32)
               + ben_ref[...])
    h = _relu(
        jnp.dot(hn_ref[...], wno1_ref[...], preferred_element_type=jnp.float32)
        + jnp.dot(en, wno2_ref[...], preferred_element_type=jnp.float32)
        + bno_ref[...])
    hn2_ref[...] = _relu(
        jnp.dot(h, wns_ref[...], preferred_element_type=jnp.float32)
        + bns_ref[...])
    a_ref[...] = jnp.dot(h, wne1_ref[...], preferred_element_type=jnp.float32)
    b_ref[...] = jnp.dot(h, wne2_ref[...], preferred_element_type=jnp.float32)


def _node_final_body(agg0_ref, agg1_ref, hn_ref, wen_ref, ben_ref,
                     wno1_ref, wno2_ref, bno_ref, h_ref):
    agg = agg0_ref[...] + agg1_ref[...]
    en = _relu(jnp.dot(agg, wen_ref[...], preferred_element_type=jnp.float32)
               + ben_ref[...])
    h_ref[...] = _relu(
        jnp.dot(hn_ref[...], wno1_ref[...], preferred_element_type=jnp.float32)
        + jnp.dot(en, wno2_ref[...], preferred_element_type=jnp.float32)
        + bno_ref[...])


def _edge_body(e_ref, p_ref, wes_ref, bes_ref, bne_ref,
               weo1_ref, weo2_ref, beo_ref, out_ref):
    ee = _relu(jnp.dot(e_ref[...], wes_ref[...],
                       preferred_element_type=jnp.float32) + bes_ref[...])
    ne = _relu(p_ref[...] + bne_ref[...])
    out_ref[...] = _relu(
        jnp.dot(ee, weo1_ref[...], preferred_element_type=jnp.float32)
        + jnp.dot(ne, weo2_ref[...], preferred_element_type=jnp.float32)
        + beo_ref[...])


def _wspec(shape):
    nd = len(shape)
    return pl.BlockSpec(shape, lambda i: (0,) * nd)


def _nspec(w):
    return pl.BlockSpec((BN, w), lambda i: (i, 0))


def _node_pre(h, wns, bns, wne1, wne2):
    nin = h.shape[1]
    return pl.pallas_call(
        _node_pre_body,
        grid=(N // BN,),
        in_specs=[_nspec(nin),
                  _wspec((nin, H)), _wspec((1, H)),
                  _wspec((nin, H)), _wspec((nin, H))],
        out_specs=[_nspec(H)] * 3,
        out_shape=[jax.ShapeDtypeStruct((N, H), jnp.float32)] * 3,
    )(h, wns, bns, wne1, wne2)


def _node_merged(agg0, agg1, hn, wen, ben, wno1, wno2, bno,
                 wns, bns, wne1, wne2):
    ein = agg0.shape[1]
    return pl.pallas_call(
        _node_merged_body,
        grid=(N // BN,),
        in_specs=[_nspec(ein), _nspec(ein), _nspec(H),
                  _wspec((ein, H)), _wspec((1, H)),
                  _wspec((H, H)), _wspec((H, H)), _wspec((1, H)),
                  _wspec((H, H)), _wspec((1, H)),
                  _wspec((H, H)), _wspec((H, H))],
        out_specs=[_nspec(H)] * 3,
        out_shape=[jax.ShapeDtypeStruct((N, H), jnp.float32)] * 3,
    )(agg0, agg1, hn, wen, ben, wno1, wno2, bno, wns, bns, wne1, wne2)


def _node_final(agg0, agg1, hn, wen, ben, wno1, wno2, bno):
    ein = agg0.shape[1]
    return pl.pallas_call(
        _node_final_body,
        grid=(N // BN,),
        in_specs=[_nspec(ein), _nspec(ein), _nspec(H),
                  _wspec((ein, H)), _wspec((1, H)),
                  _wspec((H, H)), _wspec((H, H)), _wspec((1, H))],
        out_specs=_nspec(H),
        out_shape=jax.ShapeDtypeStruct((N, H), jnp.float32),
    )(agg0, agg1, hn, wen, ben, wno1, wno2, bno)


def _edge_update(e, p, wes, bes, bne, weo1, weo2, beo):
    ein = e.shape[1]
    return pl.pallas_call(
        _edge_body,
        grid=(E // BE,),
        in_specs=[pl.BlockSpec((BE, ein), lambda i: (i, 0)),
                  pl.BlockSpec((BE, H), lambda i: (i, 0)),
                  _wspec((ein, H)), _wspec((1, H)), _wspec((1, H)),
                  _wspec((H, H)), _wspec((H, H)), _wspec((1, H))],
        out_specs=pl.BlockSpec((BE, H), lambda i: (i, 0)),
        out_shape=jax.ShapeDtypeStruct((E, H), jnp.float32),
    )(e, p, wes, bes, bne, weo1, weo2, beo)


def kernel(x, edge_index, edge_attr, params):
    src = edge_index[0]
    dst = edge_index[1]
    e = edge_attr
    n_layers = len(params)
    r1 = lambda v: v.reshape(1, -1)

    p0 = params[0]
    nin0 = x.shape[1]
    hn, a, b = _node_pre(x, p0['W_ns'], r1(p0['b_ns']),
                         p0['W_ne'][:nin0], p0['W_ne'][nin0:])

    h = None
    for l, p in enumerate(params):
        ein = e.shape[1]
        last = l == n_layers - 1

        zeros = jnp.zeros((N, ein), jnp.float32)
        aggp = _make_scatter(ein)(e, dst, zeros)

        if not last:
            psum = _make_gather()(a, b, src, dst)
            e = _edge_update(e, psum, p['W_es'], r1(p['b_es']),
                             r1(p['b_ne']), p['W_eo'][:H], p['W_eo'][H:],
                             r1(p['b_eo']))
            pn = params[l + 1]
            hn, a, b = _node_merged(
                aggp[0], aggp[1], hn, p['W_en'], r1(p['b_en']),
                p['W_no'][:H], p['W_no'][H:], r1(p['b_no']),
                pn['W_ns'], r1(pn['b_ns']), pn['W_ne'][:H], pn['W_ne'][H:])
        else:
            h = _node_final(aggp[0], aggp[1], hn, p['W_en'], r1(p['b_en']),
                            p['W_no'][:H], p['W_no'][H:], r1(p['b_no']))
    return h


# speedup vs baseline: 3.7336x; 1.0111x over previous
import functools

import jax
import jax.numpy as jnp
from jax import lax
from jax.experimental import pallas as pl
from jax.experimental.pallas import tpu as pltpu
from jax.experimental.pallas import tpu_sc as plsc

N = 10000
E = 320000
H = 64

NC = 2
NS = 16
NW = NC * NS
EW = E // NW
ROWS_PER_TILE = N // NS


GCH = 400
GNCHUNK = EW // GCH


@functools.lru_cache(maxsize=None)
def _make_gather():
    mesh = plsc.VectorSubcoreMesh(core_axis_name="c", subcore_axis_name="s")

    @functools.partial(
        pl.kernel,
        mesh=mesh,
        out_type=jax.ShapeDtypeStruct((E, H), jnp.float32),
        scratch_types=[pltpu.VMEM((EW,), jnp.int32),
                       pltpu.VMEM((EW,), jnp.int32),
                       pltpu.VMEM((GCH, H), jnp.float32),
                       pltpu.VMEM((GCH, H), jnp.float32),
                       pltpu.VMEM((GCH, H), jnp.float32),
                       pltpu.VMEM((GCH, H), jnp.float32),
                       pltpu.SemaphoreType.DMA,
                       pltpu.SemaphoreType.DMA,
                       pltpu.SemaphoreType.DMA,
                       pltpu.SemaphoreType.DMA,
                       pltpu.SemaphoreType.DMA,
                       pltpu.SemaphoreType.DMA],
        compiler_params=pltpu.CompilerParams(use_tc_tiling_on_sc=False),
    )
    def gather_k(a_hbm, b_hbm, src_hbm, dst_hbm, p_hbm,
                 idxs, idxd, bufa0, bufb0, bufa1, bufb1,
                 sa0, sb0, sa1, sb1, sw0, sw1):
        wid = lax.axis_index("s") * NC + lax.axis_index("c")
        base0 = wid * EW
        pltpu.sync_copy(src_hbm.at[pl.ds(base0, EW)], idxs)
        pltpu.sync_copy(dst_hbm.at[pl.ds(base0, EW)], idxd)

        def vadd(bufa, bufb):
            @pl.loop(0, GCH)
            def _(r):
                for jj in range(H // 16):
                    sl = pl.ds(jj * 16, 16)
                    plsc.addupdate(bufa.at[r, sl], bufb[r, sl])

        @pl.loop(0, GNCHUNK - (GNCHUNK % 2), step=2)
        def _(i):
            o0 = i * GCH
            o1 = o0 + GCH
            da0 = pltpu.async_copy(a_hbm.at[idxs.at[pl.ds(o0, GCH)]],
                                   bufa0, sa0)
            db0 = pltpu.async_copy(b_hbm.at[idxd.at[pl.ds(o0, GCH)]],
                                   bufb0, sb0)
            da1 = pltpu.async_copy(a_hbm.at[idxs.at[pl.ds(o1, GCH)]],
                                   bufa1, sa1)
            db1 = pltpu.async_copy(b_hbm.at[idxd.at[pl.ds(o1, GCH)]],
                                   bufb1, sb1)
            da0.wait()
            db0.wait()
            vadd(bufa0, bufb0)
            w0 = pltpu.async_copy(bufa0, p_hbm.at[pl.ds(base0 + o0, GCH)],
                                  sw0)
            da1.wait()
            db1.wait()
            vadd(bufa1, bufb1)
            w1 = pltpu.async_copy(bufa1, p_hbm.at[pl.ds(base0 + o1, GCH)],
                                  sw1)
            w0.wait()
            w1.wait()

        if GNCHUNK % 2:
            ot = (GNCHUNK - 1) * GCH
            da0 = pltpu.async_copy(a_hbm.at[idxs.at[pl.ds(ot, GCH)]],
                                   bufa0, sa0)
            db0 = pltpu.async_copy(b_hbm.at[idxd.at[pl.ds(ot, GCH)]],
                                   bufb0, sb0)
            da0.wait()
            db0.wait()
            vadd(bufa0, bufb0)
            pltpu.sync_copy(bufa0, p_hbm.at[pl.ds(base0 + ot, GCH)])

    return gather_k


SCH = 400


@functools.lru_cache(maxsize=None)
def _make_scatter(d):
    mesh = plsc.VectorSubcoreMesh(core_axis_name="c", subcore_axis_name="s")
    epc = E // NC
    ept = epc // NS
    nchunk = ept // SCH

    @functools.partial(
        pl.kernel,
        mesh=mesh,
        out_type=jax.ShapeDtypeStruct((NC, N, d), jnp.float32),
        scratch_types=[pltpu.VMEM((SCH,), jnp.int32),
                       pltpu.VMEM((SCH,), jnp.int32),
                       pltpu.VMEM((SCH, d), jnp.float32),
                       pltpu.VMEM((SCH, d), jnp.float32),
                       pltpu.VMEM_SHARED((N, d), jnp.float32),
                       pltpu.SemaphoreType.DMA,
                       pltpu.SemaphoreType.DMA,
                       pltpu.SemaphoreType.DMA,
                       pltpu.SemaphoreType.DMA],
        compiler_params=pltpu.CompilerParams(use_tc_tiling_on_sc=False),
    )
    def scatter_k(e_hbm, dst_hbm, zeros_hbm, out_hbm,
                  idx0, idx1, rows0, rows1, acc_sh, si0, si1, sr0, sr1):
        c = lax.axis_index("c")
        s = lax.axis_index("s")
        rbase = s * ROWS_PER_TILE
        pltpu.sync_copy(zeros_hbm.at[pl.ds(rbase, ROWS_PER_TILE)],
                        acc_sh.at[pl.ds(rbase, ROWS_PER_TILE)])
        plsc.subcore_barrier()

        ebase0 = c * epc + s * ept

        @pl.loop(0, nchunk - (nchunk % 2), step=2)
        def _(i):
            b0 = ebase0 + i * SCH
            b1 = b0 + SCH
            di0 = pltpu.async_copy(dst_hbm.at[pl.ds(b0, SCH)], idx0, si0)
            dr0 = pltpu.async_copy(e_hbm.at[pl.ds(b0, SCH)], rows0, sr0)
            di1 = pltpu.async_copy(dst_hbm.at[pl.ds(b1, SCH)], idx1, si1)
            dr1 = pltpu.async_copy(e_hbm.at[pl.ds(b1, SCH)], rows1, sr1)
            di0.wait()
            dr0.wait()
            pltpu.sync_copy(rows0, acc_sh.at[idx0], add=True)
            di1.wait()
            dr1.wait()
            pltpu.sync_copy(rows1, acc_sh.at[idx1], add=True)

        if nchunk % 2:
            bt = ebase0 + (nchunk - 1) * SCH
            di0 = pltpu.async_copy(dst_hbm.at[pl.ds(bt, SCH)], idx0, si0)
            dr0 = pltpu.async_copy(e_hbm.at[pl.ds(bt, SCH)], rows0, sr0)
            di0.wait()
            dr0.wait()
            pltpu.sync_copy(rows0, acc_sh.at[idx0], add=True)

        plsc.subcore_barrier()
        pltpu.sync_copy(acc_sh.at[pl.ds(rbase, ROWS_PER_TILE)],
                        out_hbm.at[c, pl.ds(rbase, ROWS_PER_TILE)])

    return scatter_k


BN = 2000
BE = 8000


def _relu(v):
    return jnp.maximum(v, 0.0)


def _node_pre_body(h_ref, wns_ref, bns_ref, wne1_ref, wne2_ref,
                   hn_ref, a_ref, b_ref):
    h = h_ref[...]
    hn_ref[...] = _relu(
        jnp.dot(h, wns_ref[...], preferred_element_type=jnp.float32)
        + bns_ref[...])
    a_ref[...] = jnp.dot(h, wne1_ref[...], preferred_element_type=jnp.float32)
    b_ref[...] = jnp.dot(h, wne2_ref[...], preferred_element_type=jnp.float32)


def _node_merged_body(agg0_ref, agg1_ref, hn_ref, wen_ref, ben_ref,
                      wno1_ref, wno2_ref, bno_ref,
                      wns_ref, bns_ref, wne1_ref, wne2_ref,
                      hn2_ref, a_ref, b_ref):
    agg = agg0_ref[...] + agg1_ref[...]
    en = _relu(jnp.dot(agg, wen_ref[...], preferred_element_type=jnp.float32)
               + ben_ref[...])
    h = _relu(
        jnp.dot(hn_ref[...], wno1_ref[...], preferred_element_type=jnp.float32)
        + jnp.dot(en, wno2_ref[...], preferred_element_type=jnp.float32)
        + bno_ref[...])
    hn2_ref[...] = _relu(
        jnp.dot(h, wns_ref[...], preferred_element_type=jnp.float32)
        + bns_ref[...])
    a_ref[...] = jnp.dot(h, wne1_ref[...], preferred_element_type=jnp.float32)
    b_ref[...] = jnp.dot(h, wne2_ref[...], preferred_element_type=jnp.float32)


def _node_final_body(agg0_ref, agg1_ref, hn_ref, wen_ref, ben_ref,
                     wno1_ref, wno2_ref, bno_ref, h_ref):
    agg = agg0_ref[...] + agg1_ref[...]
    en = _relu(jnp.dot(agg, wen_ref[...], preferred_element_type=jnp.float32)
               + ben_ref[...])
    h_ref[...] = _relu(
        jnp.dot(hn_ref[...], wno1_ref[...], preferred_element_type=jnp.float32)
        + jnp.dot(en, wno2_ref[...], preferred_element_type=jnp.float32)
        + bno_ref[...])


def _edge_body(e_ref, p_ref, wes_ref, bes_ref, bne_ref,
               weo1_ref, weo2_ref, beo_ref, out_ref):
    ee = _relu(jnp.dot(e_ref[...], wes_ref[...],
                       preferred_element_type=jnp.float32) + bes_ref[...])
    ne = _relu(p_ref[...] + bne_ref[...])
    out_ref[...] = _relu(
        jnp.dot(ee, weo1_ref[...], preferred_element_type=jnp.float32)
        + jnp.dot(ne, weo2_ref[...], preferred_element_type=jnp.float32)
        + beo_ref[...])


def _wspec(shape):
    nd = len(shape)
    return pl.BlockSpec(shape, lambda i: (0,) * nd)


def _nspec(w):
    return pl.BlockSpec((BN, w), lambda i: (i, 0))


def _node_pre(h, wns, bns, wne1, wne2):
    nin = h.shape[1]
    return pl.pallas_call(
        _node_pre_body,
        grid=(N // BN,),
        in_specs=[_nspec(nin),
                  _wspec((nin, H)), _wspec((1, H)),
                  _wspec((nin, H)), _wspec((nin, H))],
        out_specs=[_nspec(H)] * 3,
        out_shape=[jax.ShapeDtypeStruct((N, H), jnp.float32)] * 3,
    )(h, wns, bns, wne1, wne2)


def _node_merged(agg0, agg1, hn, wen, ben, wno1, wno2, bno,
                 wns, bns, wne1, wne2):
    ein = agg0.shape[1]
    return pl.pallas_call(
        _node_merged_body,
        grid=(N // BN,),
        in_specs=[_nspec(ein), _nspec(ein), _nspec(H),
                  _wspec((ein, H)), _wspec((1, H)),
                  _wspec((H, H)), _wspec((H, H)), _wspec((1, H)),
                  _wspec((H, H)), _wspec((1, H)),
                  _wspec((H, H)), _wspec((H, H))],
        out_specs=[_nspec(H)] * 3,
        out_shape=[jax.ShapeDtypeStruct((N, H), jnp.float32)] * 3,
    )(agg0, agg1, hn, wen, ben, wno1, wno2, bno, wns, bns, wne1, wne2)


def _node_final(agg0, agg1, hn, wen, ben, wno1, wno2, bno):
    ein = agg0.shape[1]
    return pl.pallas_call(
        _node_final_body,
        grid=(N // BN,),
        in_specs=[_nspec(ein), _nspec(ein), _nspec(H),
                  _wspec((ein, H)), _wspec((1, H)),
                  _wspec((H, H)), _wspec((H, H)), _wspec((1, H))],
        out_specs=_nspec(H),
        out_shape=jax.ShapeDtypeStruct((N, H), jnp.float32),
    )(agg0, agg1, hn, wen, ben, wno1, wno2, bno)


def _edge_update(e, p, wes, bes, bne, weo1, weo2, beo):
    ein = e.shape[1]
    return pl.pallas_call(
        _edge_body,
        grid=(E // BE,),
        in_specs=[pl.BlockSpec((BE, ein), lambda i: (i, 0)),
                  pl.BlockSpec((BE, H), lambda i: (i, 0)),
                  _wspec((ein, H)), _wspec((1, H)), _wspec((1, H)),
                  _wspec((H, H)), _wspec((H, H)), _wspec((1, H))],
        out_specs=pl.BlockSpec((BE, H), lambda i: (i, 0)),
        out_shape=jax.ShapeDtypeStruct((E, H), jnp.float32),
    )(e, p, wes, bes, bne, weo1, weo2, beo)


def kernel(x, edge_index, edge_attr, params):
    src = edge_index[0]
    dst = edge_index[1]
    e = edge_attr
    n_layers = len(params)
    r1 = lambda v: v.reshape(1, -1)

    p0 = params[0]
    nin0 = x.shape[1]
    hn, a, b = _node_pre(x, p0['W_ns'], r1(p0['b_ns']),
                         p0['W_ne'][:nin0], p0['W_ne'][nin0:])

    h = None
    for l, p in enumerate(params):
        ein = e.shape[1]
        last = l == n_layers - 1

        zeros = jnp.zeros((N, ein), jnp.float32)
        aggp = _make_scatter(ein)(e, dst, zeros)

        if not last:
            psum = _make_gather()(a, b, src, dst)
            e = _edge_update(e, psum, p['W_es'], r1(p['b_es']),
                             r1(p['b_ne']), p['W_eo'][:H], p['W_eo'][H:],
                             r1(p['b_eo']))
            pn = params[l + 1]
            hn, a, b = _node_merged(
                aggp[0], aggp[1], hn, p['W_en'], r1(p['b_en']),
                p['W_no'][:H], p['W_no'][H:], r1(p['b_no']),
                pn['W_ns'], r1(pn['b_ns']), pn['W_ne'][:H], pn['W_ne'][H:])
        else:
            h = _node_final(aggp[0], aggp[1], hn, p['W_en'], r1(p['b_en']),
                            p['W_no'][:H], p['W_no'][H:], r1(p['b_no']))
    return h
